# Initial kernel scaffold; baseline (speedup 1.0000x reference)
import jax
import jax.numpy as jnp
from jax.experimental import pallas as pl


def kernel(x, edge_index, adj_mask, W1, W2, W_out):
    raise NotImplementedError("write your pallas kernel here")


# speedup vs baseline: 2.2200x; 2.2200x over previous
import functools

import jax
import jax.numpy as jnp
from jax import lax
from jax.experimental import pallas as pl
from jax.experimental.pallas import tpu as pltpu
from jax.experimental.pallas import tpu_sc as plsc

N = 10000
D = 128
C = 8
E = 320000

NC = 2
NS = 16
CHUNK = 128
K = 80
E_TILE = CHUNK * K
E_PAD = E_TILE * NC * NS
EROWS = E_PAD // CHUNK
ACC_ROWS = 10112
ZPT = ACC_ROWS // NS


def _make_spmm(cf: bool, interpret: bool = False):
    mesh = plsc.VectorSubcoreMesh(core_axis_name="c", subcore_axis_name="s",
                                  num_cores=NC, num_subcores=NS)

    @functools.partial(
        pl.kernel,
        out_type=jax.ShapeDtypeStruct((NC * ACC_ROWS, D), jnp.float32),
        mesh=mesh,
        scratch_types=[
            pltpu.VMEM_SHARED((ACC_ROWS, D), jnp.float32),
            pltpu.VMEM((K, CHUNK), jnp.int32),
            pltpu.VMEM((K, CHUNK), jnp.int32),
            pltpu.VMEM((K, CHUNK), jnp.float32),
            pltpu.VMEM((CHUNK, D), jnp.float32),
        ],
        interpret=interpret,
    )
    def spmm(table, srcm, dstm, am, out, acc, src_v, dst_v, w_v, rows_v):
        c = lax.axis_index("c")
        s = lax.axis_index("s")
        g = c * NS + s

        def zrow(i, carry):
            for j in range(D // 16):
                rows_v[i, pl.ds(j * 16, 16)] = jnp.zeros((16,), jnp.float32)
            return carry

        lax.fori_loop(0, CHUNK, zrow, None)
        for k in range(ZPT // CHUNK):
            pltpu.sync_copy(rows_v, acc.at[pl.ds(s * ZPT + k * CHUNK, CHUNK)])
        rem = ZPT % CHUNK
        if rem:
            pltpu.sync_copy(
                rows_v.at[pl.ds(0, rem)],
                acc.at[pl.ds(s * ZPT + (ZPT // CHUNK) * CHUNK, rem)])
        plsc.subcore_barrier()

        pltpu.sync_copy(srcm.at[pl.ds(g * K, K)], src_v)
        pltpu.sync_copy(dstm.at[pl.ds(g * K, K)], dst_v)
        pltpu.sync_copy(am.at[pl.ds(g * K, K)], w_v)

        def wbody(k, carry):
            for j in range(CHUNK // 16):
                sl = pl.ds(j * 16, 16)
                a = w_v[k, sl]
                sig = 1.0 / (1.0 + jnp.exp(-a))
                wv = (1.0 - sig) if cf else sig
                w_v[k, sl] = jnp.where(src_v[k, sl] != dst_v[k, sl], wv, 1.0)
            return carry

        lax.fori_loop(0, K, wbody, None)

        def chunk_body(k, carry):
            if interpret:
                src_idx, dst_idx = src_v[k, :], dst_v[k, :]
            else:
                src_idx, dst_idx = src_v.at[k], dst_v.at[k]
            pltpu.sync_copy(table.at[src_idx], rows_v)

            def group_body(q, carry2):
                wv = w_v[k, pl.ds(q * 16, 16)]
                base = q * 16
                for l in range(16):
                    wb = jnp.full((16,), wv[l], jnp.float32)
                    e = base + l
                    for j in range(D // 16):
                        sl = pl.ds(j * 16, 16)
                        rows_v[e, sl] = rows_v[e, sl] * wb
                return carry2

            lax.fori_loop(0, CHUNK // 16, group_body, None)
            if interpret:
                def scat(e, carry2):
                    d = dst_v[k, e]
                    acc[d, :] = acc[d, :] + rows_v[e, :]
                    return carry2
                lax.fori_loop(0, CHUNK, scat, None)
            else:
                pltpu.sync_copy(rows_v, acc.at[dst_idx], add=True)
            return carry

        lax.fori_loop(0, K, chunk_body, None)
        plsc.subcore_barrier()

        pltpu.sync_copy(acc.at[pl.ds(s * ZPT, ZPT)],
                        out.at[pl.ds(c * ACC_ROWS + s * ZPT, ZPT)])

    return spmm


@functools.lru_cache(maxsize=None)
def _spmm(cf: bool):
    return _make_spmm(cf)


def _h_body(p_ref, w_ref, o_ref):
    a = p_ref[0] + p_ref[1]
    o_ref[...] = jnp.maximum(
        jnp.dot(a, w_ref[...], preferred_element_type=jnp.float32), 0.0)


_BN = 1000


def _h_layer(parts, w):
    return pl.pallas_call(
        _h_body,
        grid=(N // _BN,),
        in_specs=[pl.BlockSpec((2, _BN, D), lambda i: (0, i, 0)),
                  pl.BlockSpec((D, D), lambda i: (0, 0))],
        out_specs=pl.BlockSpec((_BN, D), lambda i: (i, 0)),
        out_shape=jax.ShapeDtypeStruct((N, D), jnp.float32),
    )(parts, w)


def _final_body(p_ref, w2_ref, wo_ref, o_ref, acc_ref):
    i = pl.program_id(0)

    @pl.when(i == 0)
    def _():
        acc_ref[...] = jnp.zeros_like(acc_ref)

    a = p_ref[0] + p_ref[1]
    h2 = jnp.maximum(
        jnp.dot(a, w2_ref[...], preferred_element_type=jnp.float32), 0.0)
    acc_ref[...] += jnp.sum(h2, axis=0, keepdims=True)

    @pl.when(i == pl.num_programs(0) - 1)
    def _():
        pooled = acc_ref[...] * (1.0 / N)
        o_ref[...] = jnp.dot(pooled, wo_ref[...],
                             preferred_element_type=jnp.float32)


def _final(parts, w2, w_out):
    return pl.pallas_call(
        _final_body,
        grid=(N // _BN,),
        in_specs=[pl.BlockSpec((2, _BN, D), lambda i: (0, i, 0)),
                  pl.BlockSpec((D, D), lambda i: (0, 0)),
                  pl.BlockSpec((D, C), lambda i: (0, 0))],
        out_specs=pl.BlockSpec((1, C), lambda i: (0, 0)),
        out_shape=jax.ShapeDtypeStruct((1, C), jnp.float32),
        scratch_shapes=[pltpu.VMEM((1, D), jnp.float32)],
    )(parts, w2, w_out)


def kernel(x, edge_index, adj_mask, W1, W2, W_out):
    src = edge_index[0]
    dst = edge_index[1]
    pad = E_PAD - E
    srcm = jnp.concatenate(
        [src, jnp.zeros((pad,), jnp.int32)]).reshape(EROWS, CHUNK)
    dstm = jnp.concatenate(
        [dst, jnp.full((pad,), N, jnp.int32)]).reshape(EROWS, CHUNK)
    am = jnp.concatenate(
        [adj_mask, jnp.zeros((pad,), jnp.float32)]).reshape(EROWS, CHUNK)

    spmm_f, spmm_cf = _spmm(False), _spmm(True)
    p1f = spmm_f(x, srcm, dstm, am).reshape(NC, ACC_ROWS, D)
    p1c = spmm_cf(x, srcm, dstm, am).reshape(NC, ACC_ROWS, D)
    h1f = _h_layer(p1f, W1)
    h1c = _h_layer(p1c, W1)
    p2f = spmm_f(h1f, srcm, dstm, am).reshape(NC, ACC_ROWS, D)
    p2c = spmm_cf(h1c, srcm, dstm, am).reshape(NC, ACC_ROWS, D)
    pred_f = _final(p2f, W2, W_out).reshape(C)
    pred_c = _final(p2c, W2, W_out).reshape(C)
    return (pred_f, pred_c)


# speedup vs baseline: 2.6630x; 1.1995x over previous
import functools

import jax
import jax.numpy as jnp
from jax import lax
from jax.experimental import pallas as pl
from jax.experimental.pallas import tpu as pltpu
from jax.experimental.pallas import tpu_sc as plsc

N = 10000
D = 128
C = 8
E = 320000

NC = 2
NS = 16
CHUNK = 64
SEGS = 4
KH = 40
K = SEGS * KH
E_TILE = CHUNK * K
E_PAD = E_TILE * NC * NS
EROWS = E_PAD // CHUNK
ACC_ROWS = 10112
ZPT = ACC_ROWS // NS


def _make_spmm(cf: bool):
    mesh = plsc.VectorSubcoreMesh(core_axis_name="c", subcore_axis_name="s",
                                  num_cores=NC, num_subcores=NS)

    @functools.partial(
        pl.kernel,
        out_type=jax.ShapeDtypeStruct((NC * ACC_ROWS, D), jnp.float32),
        mesh=mesh,
        scratch_types=[
            pltpu.VMEM_SHARED((ACC_ROWS, D), jnp.float32),
            pltpu.VMEM((KH, CHUNK), jnp.int32),
            pltpu.VMEM((KH, CHUNK), jnp.int32),
            pltpu.VMEM((KH, CHUNK), jnp.float32),
            pltpu.VMEM((CHUNK, D), jnp.float32),
            pltpu.VMEM((CHUNK, D), jnp.float32),
            pltpu.VMEM((CHUNK, D), jnp.float32),
            pltpu.SemaphoreType.DMA((3,)),
            pltpu.SemaphoreType.DMA((3,)),
        ],
    )
    def spmm(table, srcm, dstm, am, out, acc, src_v, dst_v, w_v,
             rows0, rows1, rows2, gsem, ssem):
        c_ax = lax.axis_index("c")
        s = lax.axis_index("s")
        g = c_ax * NS + s
        rows = (rows0, rows1, rows2)

        def zrow(i, carry):
            for j in range(D // 16):
                rows0[i, pl.ds(j * 16, 16)] = jnp.zeros((16,), jnp.float32)
            return carry

        lax.fori_loop(0, CHUNK, zrow, None)

        def zacc(i, carry):
            pltpu.sync_copy(rows0, acc.at[pl.ds(s * ZPT + i * CHUNK, CHUNK)])
            return carry

        lax.fori_loop(0, ZPT // CHUNK, zacc, None)
        rem = ZPT % CHUNK
        if rem:
            pltpu.sync_copy(
                rows0.at[pl.ds(0, rem)],
                acc.at[pl.ds(s * ZPT + (ZPT // CHUNK) * CHUNK, rem)])
        plsc.subcore_barrier()

        def gather_start(k, b):
            pltpu.make_async_copy(
                table.at[src_v.at[k]], rows[b], gsem.at[b]).start()

        def gather_wait(k, b):
            pltpu.make_async_copy(
                table.at[src_v.at[k]], rows[b], gsem.at[b]).wait()

        def scatter_start(k, b):
            pltpu.make_async_copy(
                rows[b], acc.at[dst_v.at[k]], ssem.at[b]).start(add=True)

        def scatter_wait(k, b):
            pltpu.make_async_copy(
                rows[b], acc.at[dst_v.at[k]], ssem.at[b]).wait()

        def compute(k, b):
            rb = rows[b]

            def group_body(q, carry):
                wv = w_v[k, pl.ds(q * 16, 16)]
                base = q * 16
                for l in range(16):
                    wb = jnp.full((16,), wv[l], jnp.float32)
                    e = base + l
                    for j in range(D // 16):
                        sl = pl.ds(j * 16, 16)
                        rb[e, sl] = rb[e, sl] * wb
                return carry

            lax.fori_loop(0, CHUNK // 16, group_body, None)

        for seg in range(SEGS):
            row0 = g * K + seg * KH
            pltpu.sync_copy(srcm.at[pl.ds(row0, KH)], src_v)
            pltpu.sync_copy(dstm.at[pl.ds(row0, KH)], dst_v)
            pltpu.sync_copy(am.at[pl.ds(row0, KH)], w_v)

            def wbody(k, carry):
                for j in range(CHUNK // 16):
                    sl = pl.ds(j * 16, 16)
                    a = w_v[k, sl]
                    sig = 1.0 / (1.0 + jnp.exp(-a))
                    wv = (1.0 - sig) if cf else sig
                    w_v[k, sl] = jnp.where(
                        src_v[k, sl] != dst_v[k, sl], wv, 1.0)
                return carry

            lax.fori_loop(0, KH, wbody, None)

            n_steady = (KH - 3) // 3
            tail0 = 1 + 3 * n_steady
            gather_start(0, 0)
            gather_start(1, 1)
            gather_wait(0, 0)
            compute(0, 0)
            scatter_start(0, 0)
            gather_start(2, 2)

            def steady(o, carry):
                for b3 in range(3):
                    k = 1 + o * 3 + b3
                    b = (1 + b3) % 3
                    gather_wait(k, b)
                    compute(k, b)
                    scatter_wait(k - 1, (b + 2) % 3)
                    scatter_start(k, b)
                    gather_start(k + 2, (b + 2) % 3)
                return carry

            lax.fori_loop(0, n_steady, steady, None)

            for k in range(tail0, KH):
                b = k % 3
                gather_wait(k, b)
                compute(k, b)
                scatter_wait(k - 1, (b + 2) % 3)
                scatter_start(k, b)
                if k + 2 < KH:
                    gather_start(k + 2, (b + 2) % 3)
            scatter_wait(KH - 1, (KH - 1) % 3)

        plsc.subcore_barrier()

        pltpu.sync_copy(acc.at[pl.ds(s * ZPT, ZPT)],
                        out.at[pl.ds(c_ax * ACC_ROWS + s * ZPT, ZPT)])

    return spmm


@functools.lru_cache(maxsize=None)
def _spmm(cf: bool):
    return _make_spmm(cf)


def _h_body(p_ref, w_ref, o_ref):
    a = p_ref[0] + p_ref[1]
    o_ref[...] = jnp.maximum(
        jnp.dot(a, w_ref[...], preferred_element_type=jnp.float32), 0.0)


_BN = 1000


def _h_layer(parts, w):
    return pl.pallas_call(
        _h_body,
        grid=(N // _BN,),
        in_specs=[pl.BlockSpec((2, _BN, D), lambda i: (0, i, 0)),
                  pl.BlockSpec((D, D), lambda i: (0, 0))],
        out_specs=pl.BlockSpec((_BN, D), lambda i: (i, 0)),
        out_shape=jax.ShapeDtypeStruct((N, D), jnp.float32),
    )(parts, w)


def _final_body(p_ref, w2_ref, wo_ref, o_ref, acc_ref):
    i = pl.program_id(0)

    @pl.when(i == 0)
    def _():
        acc_ref[...] = jnp.zeros_like(acc_ref)

    a = p_ref[0] + p_ref[1]
    h2 = jnp.maximum(
        jnp.dot(a, w2_ref[...], preferred_element_type=jnp.float32), 0.0)
    acc_ref[...] += jnp.sum(h2, axis=0, keepdims=True)

    @pl.when(i == pl.num_programs(0) - 1)
    def _():
        pooled = acc_ref[...] * (1.0 / N)
        o_ref[...] = jnp.dot(pooled, wo_ref[...],
                             preferred_element_type=jnp.float32)


def _final(parts, w2, w_out):
    return pl.pallas_call(
        _final_body,
        grid=(N // _BN,),
        in_specs=[pl.BlockSpec((2, _BN, D), lambda i: (0, i, 0)),
                  pl.BlockSpec((D, D), lambda i: (0, 0)),
                  pl.BlockSpec((D, C), lambda i: (0, 0))],
        out_specs=pl.BlockSpec((1, C), lambda i: (0, 0)),
        out_shape=jax.ShapeDtypeStruct((1, C), jnp.float32),
        scratch_shapes=[pltpu.VMEM((1, D), jnp.float32)],
    )(parts, w2, w_out)


def kernel(x, edge_index, adj_mask, W1, W2, W_out):
    src = edge_index[0]
    dst = edge_index[1]
    pad = E_PAD - E
    srcm = jnp.concatenate(
        [src, jnp.zeros((pad,), jnp.int32)]).reshape(EROWS, CHUNK)
    dstm = jnp.concatenate(
        [dst, jnp.full((pad,), N, jnp.int32)]).reshape(EROWS, CHUNK)
    am = jnp.concatenate(
        [adj_mask, jnp.zeros((pad,), jnp.float32)]).reshape(EROWS, CHUNK)

    spmm_f, spmm_cf = _spmm(False), _spmm(True)
    p1f = spmm_f(x, srcm, dstm, am).reshape(NC, ACC_ROWS, D)
    p1c = spmm_cf(x, srcm, dstm, am).reshape(NC, ACC_ROWS, D)
    h1f = _h_layer(p1f, W1)
    h1c = _h_layer(p1c, W1)
    p2f = spmm_f(h1f, srcm, dstm, am).reshape(NC, ACC_ROWS, D)
    p2c = spmm_cf(h1c, srcm, dstm, am).reshape(NC, ACC_ROWS, D)
    pred_f = _final(p2f, W2, W_out).reshape(C)
    pred_c = _final(p2c, W2, W_out).reshape(C)
    return (pred_f, pred_c)
